# baseline (speedup 1.0000x reference)
import dataclasses
import functools

import jax
import jax.numpy as jnp
from jax import lax
from jax.experimental import pallas as pl
from jax.experimental.pallas import tpu as pltpu
from jax.experimental.pallas import tpu_sc as plsc

_B = 8
_NE = 50000
_NR = 128
_E = 1600000
_H = 3

_NC, _NS, _L = 2, 16, 16
_NW = _NC * _NS
_CH = 512
_NCHUNK = 100
_NCA = 102
_EPT = _NCHUNK * _CH
_EPA = _EPT * _NW
_NEP = 50048
_SPT = _NEP // _NS
_NV = _CH * _B // _L
_D = 4


def _make_hop():
    mesh = plsc.VectorSubcoreMesh(core_axis_name="c", subcore_axis_name="s")
    cp = pltpu.CompilerParams()
    for fld, val in (("needs_layout_passes", False),
                     ("use_tc_tiling_on_sc", False)):
        if fld in pltpu.CompilerParams.__dataclass_fields__:
            cp = dataclasses.replace(cp, **{fld: val})

    @functools.partial(
        pl.kernel,
        out_type=jax.ShapeDtypeStruct((_NC, _NEP, _B), jnp.float32),
        mesh=mesh,
        compiler_params=cp,
        scratch_types=[
            pltpu.VMEM_SHARED((_NEP, _B), jnp.float32),
            pltpu.VMEM((_NCHUNK, _CH), jnp.int32),
            pltpu.VMEM((_NR + _B, _B), jnp.float32),
            pltpu.VMEM((_D, _CH), jnp.int32),
            pltpu.VMEM((_D, _CH), jnp.int32),
            pltpu.VMEM((_D, _CH, _B), jnp.float32),
            pltpu.VMEM((_D, _CH, _B), jnp.float32),
            [pltpu.SemaphoreType.DMA] * _D,
            [pltpu.SemaphoreType.DMA] * _D,
            [pltpu.SemaphoreType.DMA] * _D,
        ],
    )
    def hop(e_hbm, r_hbm, head_hbm, rel_hbm, tail_hbm, z_hbm, out_hbm,
            acc, tails, rtab, hbuf, rbuf, ebuf, mbuf, semA, semB, semS):
        cid = lax.axis_index("c")
        sid = lax.axis_index("s")
        wid = cid * _NS + sid
        crow = wid * _NCA

        row0 = sid * _SPT
        pltpu.sync_copy(z_hbm.at[pl.ds(row0, _SPT)],
                        acc.at[pl.ds(row0, _SPT)])
        pltpu.sync_copy(tail_hbm.at[pl.ds(crow, _NCHUNK)], tails)
        pltpu.sync_copy(r_hbm, rtab)
        plsc.subcore_barrier()

        ii = lax.broadcasted_iota(jnp.int32, (_L,), 0)
        half = lax.shift_right_logical(ii, 3)
        cols = lax.bitwise_and(ii, 7)

        def issue_a(i, k):
            pltpu.async_copy(head_hbm.at[crow + i], hbuf.at[k], semA[k])
            pltpu.async_copy(rel_hbm.at[crow + i], rbuf.at[k], semA[k])

        def wait_a(k):
            pltpu.make_async_copy(head_hbm.at[crow], hbuf.at[k],
                                  semA[k]).wait()
            pltpu.make_async_copy(rel_hbm.at[crow], rbuf.at[k],
                                  semA[k]).wait()

        def issue_b(k):
            pltpu.async_copy(e_hbm.at[hbuf.at[k]], ebuf.at[k], semB[k])

        def wait_b(k):
            pltpu.make_async_copy(e_hbm.at[hbuf.at[k]], ebuf.at[k],
                                  semB[k]).wait()

        def issue_s(i, k):
            pltpu.async_copy(mbuf.at[k], acc.at[tails.at[i]], semS[k],
                             add=True)

        def wait_s(i, k):
            pltpu.make_async_copy(mbuf.at[k], acc.at[tails.at[i]],
                                  semS[k]).wait()

        def compute(k):
            @plsc.parallel_loop(0, _NV, unroll=8)
            def _mul(v):
                rows = half + 2 * v
                ev = plsc.load_gather(ebuf.at[k], [rows, cols])
                rr = plsc.load_gather(rbuf.at[k], [rows])
                rv = plsc.load_gather(rtab, [rr, cols])
                plsc.store_scatter(mbuf.at[k], [rows, cols], ev * rv)

        issue_a(0, 0)
        issue_a(1, 1)
        for i in range(8):
            k = i % _D
            wait_a(k)
            issue_b(k)
            if i >= 6:
                wait_s(i - 6, (i - 6) % _D)
            if i >= 2:
                wait_b((i - 2) % _D)
                compute((i - 2) % _D)
                issue_s(i - 2, (i - 2) % _D)
            issue_a(i + 2, (i + 2) % _D)

        @pl.loop(2, _NCHUNK // _D)
        def _quad(g):
            for b in range(_D):
                i = _D * g + b
                k = b
                k2 = (b + 2) % _D
                wait_a(k)
                issue_b(k)
                wait_s(i - 6, k2)
                wait_b(k2)
                compute(k2)
                issue_s(i - 2, k2)
                issue_a(i + 2, k2)

        n = _NCHUNK
        wait_s(n - 6, (n - 2) % _D)
        wait_b((n - 2) % _D)
        compute((n - 2) % _D)
        issue_s(n - 2, (n - 2) % _D)
        wait_s(n - 5, (n - 1) % _D)
        wait_b((n - 1) % _D)
        compute((n - 1) % _D)
        issue_s(n - 1, (n - 1) % _D)
        for j in range(n - 4, n):
            wait_s(j, j % _D)
        wait_a(n % _D)
        wait_a((n + 1) % _D)

        plsc.subcore_barrier()
        pltpu.sync_copy(acc.at[pl.ds(row0, _SPT)],
                        out_hbm.at[cid, pl.ds(row0, _SPT)])

    return hop


def _normalize(w2):
    def body(w_ref, o_ref):
        w = w_ref[0] + w_ref[1]
        col = jnp.sum(w, axis=0, keepdims=True)
        i = lax.broadcasted_iota(jnp.int32, (128, 128), 0)
        j = lax.broadcasted_iota(jnp.int32, (128, 128), 1)
        fold = jnp.where((i % _B) == (j % _B), 1.0, 0.0)
        s_tiled = jnp.dot(col, fold,
                          preferred_element_type=jnp.float32)
        o_ref[...] = w / (s_tiled + 1e-6)

    return pl.pallas_call(
        body,
        out_shape=jax.ShapeDtypeStruct((_NEP * _B // 128, 128), jnp.float32),
    )(w2)


_hop = _make_hop()


def _edges2d(x, fill):
    xp = jnp.concatenate([x, jnp.full((_EPA - _E,), fill, x.dtype)])
    x2 = xp.reshape(_NW, _NCHUNK, _CH)
    dummy = jnp.full((_NW, _NCA - _NCHUNK, _CH), fill, x.dtype)
    return jnp.concatenate([x2, dummy], axis=1).reshape(_NW * _NCA, _CH)


def kernel(init_ent, rels_seq, head, rel, tail):
    e0t = jnp.pad(init_ent.T, ((0, _NEP - _NE), (0, 0)))
    rt = jnp.transpose(rels_seq, (1, 2, 0))
    rt = jnp.pad(rt, ((0, 0), (0, 8), (0, 0)))
    head2 = _edges2d(head, 0)
    rel2 = _edges2d(rel, _NR)
    tail2 = _edges2d(tail, 0)
    zeros = jnp.zeros((_NEP, _B), jnp.float32)

    et = e0t
    outs = []
    for h in range(_H):
        w2 = _hop(et, rt[h], head2, rel2, tail2, zeros)
        et_flat = _normalize(w2.reshape(2, _NEP * _B // 128, 128))
        et = et_flat.reshape(_NEP, _B)
        outs.append(et[:_NE])

    walks = jnp.transpose(jnp.stack(outs, axis=0), (2, 0, 1))
    return jnp.concatenate([init_ent[:, None, :], walks], axis=1)


# speedup vs baseline: 1.4654x; 1.4654x over previous
import dataclasses
import functools

import jax
import jax.numpy as jnp
from jax import lax
from jax.experimental import pallas as pl
from jax.experimental.pallas import tpu as pltpu
from jax.experimental.pallas import tpu_sc as plsc

_B = 8
_NE = 50000
_NR = 128
_E = 1600000
_H = 3

_NC, _NS, _L = 2, 16, 16
_NW = _NC * _NS
_CH = 128
_NCHUNK = 392
_NCA = 394
_EPT = _NCHUNK * _CH
_EPA = _EPT * _NW
_NEP = 50048
_SPT = _NEP // _NS
_NV = _CH * _B // _L
_D = 4


def _make_hop():
    mesh = plsc.VectorSubcoreMesh(core_axis_name="c", subcore_axis_name="s")
    cp = pltpu.CompilerParams()
    for fld, val in (("needs_layout_passes", False),
                     ("use_tc_tiling_on_sc", False)):
        if fld in pltpu.CompilerParams.__dataclass_fields__:
            cp = dataclasses.replace(cp, **{fld: val})

    @functools.partial(
        pl.kernel,
        out_type=jax.ShapeDtypeStruct((_NC, _NEP, _B), jnp.float32),
        mesh=mesh,
        compiler_params=cp,
        scratch_types=[
            pltpu.VMEM_SHARED((_NEP, _B), jnp.float32),
            pltpu.VMEM((_NCHUNK, _CH), jnp.int32),
            pltpu.VMEM((_NR + _B, _B), jnp.float32),
            pltpu.VMEM((_D, _CH), jnp.int32),
            pltpu.VMEM((_D, _CH), jnp.int32),
            pltpu.VMEM((_D, _CH, _B), jnp.float32),
            pltpu.VMEM((_D, _CH, _B), jnp.float32),
            [pltpu.SemaphoreType.DMA] * _D,
            [pltpu.SemaphoreType.DMA] * _D,
            [pltpu.SemaphoreType.DMA] * _D,
        ],
    )
    def hop(e_hbm, r_hbm, head_hbm, rel_hbm, tail_hbm, z_hbm, out_hbm,
            acc, tails, rtab, hbuf, rbuf, ebuf, mbuf, semA, semB, semS):
        cid = lax.axis_index("c")
        sid = lax.axis_index("s")
        wid = cid * _NS + sid
        crow = wid * _NCA

        row0 = sid * _SPT
        pltpu.sync_copy(z_hbm.at[pl.ds(row0, _SPT)],
                        acc.at[pl.ds(row0, _SPT)])
        pltpu.sync_copy(tail_hbm.at[pl.ds(crow, _NCHUNK)], tails)
        pltpu.sync_copy(r_hbm, rtab)
        plsc.subcore_barrier()

        ii = lax.broadcasted_iota(jnp.int32, (_L,), 0)
        half = lax.shift_right_logical(ii, 3)
        cols = lax.bitwise_and(ii, 7)

        def issue_a(i, k):
            pltpu.async_copy(head_hbm.at[crow + i], hbuf.at[k], semA[k])
            pltpu.async_copy(rel_hbm.at[crow + i], rbuf.at[k], semA[k])

        def wait_a(k):
            pltpu.make_async_copy(head_hbm.at[crow], hbuf.at[k],
                                  semA[k]).wait()
            pltpu.make_async_copy(rel_hbm.at[crow], rbuf.at[k],
                                  semA[k]).wait()

        def issue_b(k):
            pltpu.async_copy(e_hbm.at[hbuf.at[k]], ebuf.at[k], semB[k])

        def wait_b(k):
            pltpu.make_async_copy(e_hbm.at[hbuf.at[k]], ebuf.at[k],
                                  semB[k]).wait()

        def issue_s(i, k):
            pltpu.async_copy(mbuf.at[k], acc.at[tails.at[i]], semS[k],
                             add=True)

        def wait_s(i, k):
            pltpu.make_async_copy(mbuf.at[k], acc.at[tails.at[i]],
                                  semS[k]).wait()

        def compute(k):
            @plsc.parallel_loop(0, _NV, unroll=8)
            def _mul(v):
                rows = half + 2 * v
                ev = plsc.load_gather(ebuf.at[k], [rows, cols])
                rr = plsc.load_gather(rbuf.at[k], [rows])
                rv = plsc.load_gather(rtab, [rr, cols])
                plsc.store_scatter(mbuf.at[k], [rows, cols], ev * rv)

        issue_a(0, 0)
        issue_a(1, 1)
        for i in range(8):
            k = i % _D
            wait_a(k)
            issue_b(k)
            if i >= 6:
                wait_s(i - 6, (i - 6) % _D)
            if i >= 2:
                wait_b((i - 2) % _D)
                compute((i - 2) % _D)
                issue_s(i - 2, (i - 2) % _D)
            issue_a(i + 2, (i + 2) % _D)

        @pl.loop(2, _NCHUNK // _D)
        def _quad(g):
            for b in range(_D):
                i = _D * g + b
                k = b
                k2 = (b + 2) % _D
                wait_a(k)
                issue_b(k)
                wait_s(i - 6, k2)
                wait_b(k2)
                compute(k2)
                issue_s(i - 2, k2)
                issue_a(i + 2, k2)

        n = _NCHUNK
        wait_s(n - 6, (n - 2) % _D)
        wait_b((n - 2) % _D)
        compute((n - 2) % _D)
        issue_s(n - 2, (n - 2) % _D)
        wait_s(n - 5, (n - 1) % _D)
        wait_b((n - 1) % _D)
        compute((n - 1) % _D)
        issue_s(n - 1, (n - 1) % _D)
        for j in range(n - 4, n):
            wait_s(j, j % _D)
        wait_a(n % _D)
        wait_a((n + 1) % _D)

        plsc.subcore_barrier()
        pltpu.sync_copy(acc.at[pl.ds(row0, _SPT)],
                        out_hbm.at[cid, pl.ds(row0, _SPT)])

    return hop


def _normalize(w2):
    def body(w_ref, o_ref):
        w = w_ref[0] + w_ref[1]
        col = jnp.sum(w, axis=0, keepdims=True)
        i = lax.broadcasted_iota(jnp.int32, (128, 128), 0)
        j = lax.broadcasted_iota(jnp.int32, (128, 128), 1)
        fold = jnp.where((i % _B) == (j % _B), 1.0, 0.0)
        s_tiled = jnp.dot(col, fold,
                          preferred_element_type=jnp.float32)
        o_ref[...] = w / (s_tiled + 1e-6)

    return pl.pallas_call(
        body,
        out_shape=jax.ShapeDtypeStruct((_NEP * _B // 128, 128), jnp.float32),
    )(w2)


_hop = _make_hop()


def _edges2d(x, fill):
    xp = jnp.concatenate([x, jnp.full((_EPA - _E,), fill, x.dtype)])
    x2 = xp.reshape(_NW, _NCHUNK, _CH)
    dummy = jnp.full((_NW, _NCA - _NCHUNK, _CH), fill, x.dtype)
    return jnp.concatenate([x2, dummy], axis=1).reshape(_NW * _NCA, _CH)


def kernel(init_ent, rels_seq, head, rel, tail):
    e0t = jnp.pad(init_ent.T, ((0, _NEP - _NE), (0, 0)))
    rt = jnp.transpose(rels_seq, (1, 2, 0))
    rt = jnp.pad(rt, ((0, 0), (0, 8), (0, 0)))
    head2 = _edges2d(head, 0)
    rel2 = _edges2d(rel, _NR)
    tail2 = _edges2d(tail, 0)
    zeros = jnp.zeros((_NEP, _B), jnp.float32)

    et = e0t
    outs = []
    for h in range(_H):
        w2 = _hop(et, rt[h], head2, rel2, tail2, zeros)
        et_flat = _normalize(w2.reshape(2, _NEP * _B // 128, 128))
        et = et_flat.reshape(_NEP, _B)
        outs.append(et[:_NE])

    walks = jnp.transpose(jnp.stack(outs, axis=0), (2, 0, 1))
    return jnp.concatenate([init_ent[:, None, :], walks], axis=1)
